# baseline (speedup 1.0000x reference)
import functools

import jax
import jax.numpy as jnp
from jax import lax
from jax.experimental import pallas as pl
from jax.experimental.pallas import tpu as pltpu
from jax.experimental.pallas import tpu_sc as plsc

N_USER = 60000
M_ITEM = 40000
N = N_USER + M_ITEM
E = 1600000
DIM = 32
HDIM = 16
B = 1024

NC = 2
NS = 16
L = 16

C = 512
CH = C // 128
E_PAD = 1622016
ET = E_PAD // NS
NCHUNK = ET // C
N_PAD = 100096
ROWS_T = N_PAD // NS

M_PAD = 40064
IT_T = M_PAD // NS

_mesh = plsc.VectorSubcoreMesh(core_axis_name="c", subcore_axis_name="s")


@functools.partial(
    pl.kernel,
    out_type=jax.ShapeDtypeStruct((2 * N_PAD, HDIM), jnp.float32),
    mesh=_mesh,
    scratch_types=[
        pltpu.VMEM((3, CH, 3, 128), jnp.int32),
        pltpu.VMEM((3, C, HDIM), jnp.float32),
        pltpu.VMEM_SHARED((N_PAD, HDIM), jnp.float32),
    ] + [pltpu.SemaphoreType.DMA] * 9,
    compiler_params=pltpu.CompilerParams(use_tc_tiling_on_sc=False, needs_layout_passes=False),
)
def _sc_layer(emb_in, packed, zrows, emb_out,
              pk_v, rows_v, acc,
              se0, se1, se2, sg0, sg1, sg2, ss0, ss1, ss2):
    c = lax.axis_index("c")
    s = lax.axis_index("s")
    tab_base = c * N_PAD
    sem_e = (se0, se1, se2)
    sem_g = (sg0, sg1, sg2)
    sem_s = (ss0, ss1, ss2)

    pltpu.sync_copy(zrows, acc.at[pl.ds(s * ROWS_T, ROWS_T)])
    plsc.subcore_barrier()

    ebase = s * (ET // 128)

    def load_edges(k, b):
        off = ebase + k * CH
        pltpu.async_copy(packed.at[pl.ds(off, CH)], pk_v.at[b], sem_e[b])

    def drain_edges(b):
        pltpu.make_async_copy(packed.at[pl.ds(0, CH)], pk_v.at[b],
                              sem_e[b]).wait()

    def adjust_src(b):
        @plsc.parallel_loop(0, C // L, unroll=8)
        def _adj(q):
            j = q // (128 // L)
            t = q % (128 // L)
            sl = pl.ds(t * L, L)
            pk_v[b, j, 0, sl] = pk_v[b, j, 0, sl] + tab_base

    def fire_gathers(b):
        for j in range(CH):
            pltpu.async_copy(emb_in.at[pk_v.at[b, j, 0]],
                             rows_v.at[b, pl.ds(j * 128, 128)], sem_g[b])

    def drain_gathers(b):
        pltpu.make_async_copy(emb_in.at[pl.ds(0, C)], rows_v.at[b],
                              sem_g[b]).wait()

    def scale_rows(b):
        @plsc.parallel_loop(0, C // L, unroll=2)
        def _scale(g):
            j = g // (128 // L)
            t = g % (128 // L)
            vv = plsc.bitcast(pk_v[b, j, 2, pl.ds(t * L, L)], jnp.float32)
            base = g * L
            for i in range(L):
                rows_v[b, base + i] = rows_v[b, base + i] * vv[i]

    def fire_scatters(b):
        for j in range(CH):
            pltpu.async_copy(rows_v.at[b, pl.ds(j * 128, 128)],
                             acc.at[pk_v.at[b, j, 1]], sem_s[b], add=True)

    def drain_scatters(b):
        pltpu.make_async_copy(rows_v.at[b], acc.at[pl.ds(0, C)],
                              sem_s[b]).wait()

    load_edges(0, 0)
    drain_edges(0)
    adjust_src(0)
    fire_gathers(0)
    load_edges(1, 1)
    drain_edges(1)
    adjust_src(1)
    fire_gathers(1)

    def outer_body(k3, carry):
        for b in (0, 1, 2):
            k = k3 * 3 + b
            b1 = (b + 1) % 3
            b2 = (b + 2) % 3

            @pl.when(k >= 1)
            def _():
                drain_scatters(b2)

            @pl.when(k < NCHUNK - 2)
            def _():
                load_edges(k + 2, b2)

            drain_gathers(b)
            scale_rows(b)
            fire_scatters(b)

            @pl.when(k < NCHUNK - 2)
            def _():
                drain_edges(b2)
                adjust_src(b2)
                fire_gathers(b2)
        return carry

    lax.fori_loop(0, NCHUNK // 3, outer_body, 0)
    drain_scatters((NCHUNK - 1) % 3)
    plsc.subcore_barrier()

    pltpu.sync_copy(acc.at[pl.ds(s * ROWS_T, ROWS_T)],
                    emb_out.at[pl.ds(tab_base + s * ROWS_T, ROWS_T)])


_UB = B // NS


@functools.partial(
    pl.kernel,
    out_type=(jax.ShapeDtypeStruct((B, HDIM), jnp.float32),
              jax.ShapeDtypeStruct((B, HDIM), jnp.float32)),
    mesh=_mesh,
    scratch_types=[
        pltpu.VMEM((1, _UB), jnp.int32),
        pltpu.VMEM((_UB, HDIM), jnp.float32),
        pltpu.VMEM((_UB, HDIM), jnp.float32),
        pltpu.SemaphoreType.DMA,
    ],
    compiler_params=pltpu.CompilerParams(use_tc_tiling_on_sc=False, needs_layout_passes=False),
)
def _sc_users(e0, e1, e2, e3, inst2, out_lo, out_hi,
              idx_v, rows_v, uacc_v, sem):
    c = lax.axis_index("c")
    s = lax.axis_index("s")
    pltpu.sync_copy(inst2.at[s], idx_v)

    for t in range(_UB // L):
        sl = pl.ds(t * L, L)
        idx_v[0, sl] = idx_v[0, sl] + c * N_PAD

    pltpu.async_copy(e0.at[idx_v.at[0]], uacc_v, sem).wait()
    for e in (e1, e2, e3):
        pltpu.async_copy(e.at[idx_v.at[0]], rows_v, sem).wait()

        @plsc.parallel_loop(0, _UB, unroll=8)
        def _acc(r):
            uacc_v[r] = uacc_v[r] + rows_v[r]

    @plsc.parallel_loop(0, _UB, unroll=8)
    def _mean(r):
        uacc_v[r] = uacc_v[r] * 0.25

    @pl.when(c == 0)
    def _():
        pltpu.sync_copy(uacc_v, out_lo.at[pl.ds(s * _UB, _UB)])

    @pl.when(c == 1)
    def _():
        pltpu.sync_copy(uacc_v, out_hi.at[pl.ds(s * _UB, _UB)])


@functools.partial(
    pl.kernel,
    out_type=(jax.ShapeDtypeStruct((M_PAD, HDIM), jnp.float32),
              jax.ShapeDtypeStruct((M_PAD, HDIM), jnp.float32)),
    mesh=_mesh,
    scratch_types=[
        pltpu.VMEM((IT_T, HDIM), jnp.float32),
        pltpu.VMEM((IT_T, HDIM), jnp.float32),
    ],
    compiler_params=pltpu.CompilerParams(use_tc_tiling_on_sc=False, needs_layout_passes=False),
)
def _sc_items(e0, e1, e2, e3, out_lo, out_hi, iacc_v, in_v):
    c = lax.axis_index("c")
    s = lax.axis_index("s")
    base = c * N_PAD + N_USER + s * IT_T

    pltpu.sync_copy(e0.at[pl.ds(base, IT_T)], iacc_v)
    for e in (e1, e2, e3):
        pltpu.sync_copy(e.at[pl.ds(base, IT_T)], in_v)

        @plsc.parallel_loop(0, IT_T, unroll=8)
        def _acc(r):
            iacc_v[r] = iacc_v[r] + in_v[r]

    @plsc.parallel_loop(0, IT_T, unroll=8)
    def _mean(r):
        iacc_v[r] = iacc_v[r] * 0.25

    @pl.when(c == 0)
    def _():
        pltpu.sync_copy(iacc_v, out_lo.at[pl.ds(s * IT_T, IT_T)])

    @pl.when(c == 1)
    def _():
        pltpu.sync_copy(iacc_v, out_hi.at[pl.ds(s * IT_T, IT_T)])


@functools.partial(
    pl.kernel,
    out_type=jax.ShapeDtypeStruct((2 * N_PAD, HDIM), jnp.float32),
    mesh=_mesh,
    scratch_types=[],
    compiler_params=pltpu.CompilerParams(use_tc_tiling_on_sc=False, needs_layout_passes=False),
)
def _sc_stack(user_emb, item_emb, zr96, e0_out):
    c = lax.axis_index("c")
    s = lax.axis_index("s")
    half = c * N_PAD
    csl = pl.ds(c * HDIM, HDIM)
    for t in range(16):
        @pl.when(s == t)
        def _():
            for j in (t, t + 16):
                if j < 15:
                    a = j * 4000
                    pltpu.sync_copy(user_emb.at[pl.ds(a, 4000), csl],
                                    e0_out.at[pl.ds(half + a, 4000)])
                elif j < 25:
                    a = (j - 15) * 4000
                    pltpu.sync_copy(item_emb.at[pl.ds(a, 4000), csl],
                                    e0_out.at[pl.ds(half + N_USER + a, 4000)])
                elif j == 25:
                    pltpu.sync_copy(zr96,
                                    e0_out.at[pl.ds(half + N, N_PAD - N)])


BI = 2048


def _mm_body(il_ref, ih_ref, ul_ref, uh_ref, o_ref):
    acc = lax.dot_general(il_ref[...], ul_ref[...], (((1,), (1,)), ((), ())),
                          preferred_element_type=jnp.float32)
    acc += lax.dot_general(ih_ref[...], uh_ref[...], (((1,), (1,)), ((), ())),
                           preferred_element_type=jnp.float32)
    o_ref[...] = 1.0 / (1.0 + jnp.exp(-acc))


_ratings_call = pl.pallas_call(
    _mm_body,
    grid=((M_ITEM + BI - 1) // BI,),
    in_specs=[
        pl.BlockSpec((BI, HDIM), lambda i: (i, 0)),
        pl.BlockSpec((BI, HDIM), lambda i: (i, 0)),
        pl.BlockSpec((B, HDIM), lambda i: (0, 0)),
        pl.BlockSpec((B, HDIM), lambda i: (0, 0)),
    ],
    out_specs=pl.BlockSpec((BI, B), lambda i: (i, 0)),
    out_shape=jax.ShapeDtypeStruct((M_ITEM, B), jnp.float32),
)


def kernel(instances, edge_index, edge_vals, user_emb, item_emb):
    src = edge_index[0].astype(jnp.int32)
    dst = edge_index[1].astype(jnp.int32)
    valbits = lax.bitcast_convert_type(edge_vals.astype(jnp.float32),
                                       jnp.int32)

    pad = E_PAD - E
    zpad_e = jnp.zeros((pad,), jnp.int32)

    def groups(a):
        return jnp.concatenate([a, zpad_e]).reshape(-1, 128)

    packed = jnp.stack([groups(src), groups(dst), groups(valbits)], axis=1)

    zr96 = jnp.zeros((N_PAD - N, HDIM), jnp.float32)
    e0 = _sc_stack(user_emb.astype(jnp.float32), item_emb.astype(jnp.float32),
                   zr96)

    zrows = jnp.zeros((ROWS_T, HDIM), jnp.float32)
    e1 = _sc_layer(e0, packed, zrows)
    e2 = _sc_layer(e1, packed, zrows)
    e3 = _sc_layer(e2, packed, zrows)

    inst2 = instances.astype(jnp.int32).reshape(NS, 1, _UB)
    u_lo, u_hi = _sc_users(e0, e1, e2, e3, inst2)
    items_lo, items_hi = _sc_items(e0, e1, e2, e3)

    return _ratings_call(items_lo, items_hi, u_lo, u_hi).T


# speedup vs baseline: 1.4432x; 1.4432x over previous
import functools

import jax
import jax.numpy as jnp
from jax import lax
from jax.experimental import pallas as pl
from jax.experimental.pallas import tpu as pltpu
from jax.experimental.pallas import tpu_sc as plsc

N_USER = 60000
M_ITEM = 40000
N = N_USER + M_ITEM
E = 1600000
DIM = 32
HDIM = 16
B = 1024

NC = 2
NS = 16
L = 16

C = 512
CH = C // 128
E_PAD = 1622016
ET = E_PAD // NS
NCHUNK = ET // C
N_PAD = 100096
ROWS_T = N_PAD // NS

M_PAD = 40064
IT_T = M_PAD // NS

_mesh = plsc.VectorSubcoreMesh(core_axis_name="c", subcore_axis_name="s")


@functools.partial(
    pl.kernel,
    out_type=jax.ShapeDtypeStruct((2 * N_PAD, HDIM), jnp.float32),
    mesh=_mesh,
    scratch_types=[
        pltpu.VMEM((3, CH, 3, 128), jnp.int32),
        pltpu.VMEM((3, C, HDIM), jnp.float32),
        pltpu.VMEM_SHARED((N_PAD, HDIM), jnp.float32),
    ] + [pltpu.SemaphoreType.DMA] * 9,
    compiler_params=pltpu.CompilerParams(use_tc_tiling_on_sc=False, needs_layout_passes=False),
)
def _sc_layer(emb_in, packed, zrows, emb_out,
              pk_v, rows_v, acc,
              se0, se1, se2, sg0, sg1, sg2, ss0, ss1, ss2):
    c = lax.axis_index("c")
    s = lax.axis_index("s")
    tab_base = c * N_PAD
    sem_e = (se0, se1, se2)
    sem_g = (sg0, sg1, sg2)
    sem_s = (ss0, ss1, ss2)

    pltpu.sync_copy(zrows, acc.at[pl.ds(s * ROWS_T, ROWS_T)])
    plsc.subcore_barrier()

    ebase = s * (ET // 128)

    def load_edges(k, b):
        off = ebase + k * CH
        pltpu.async_copy(packed.at[pl.ds(off, CH)], pk_v.at[b], sem_e[b])

    def drain_edges(b):
        pltpu.make_async_copy(packed.at[pl.ds(0, CH)], pk_v.at[b],
                              sem_e[b]).wait()

    def adjust_src(b):
        @plsc.parallel_loop(0, C // L, unroll=8)
        def _adj(q):
            j = q // (128 // L)
            t = q % (128 // L)
            sl = pl.ds(t * L, L)
            pk_v[b, j, 0, sl] = pk_v[b, j, 0, sl] + tab_base

    def fire_gathers(b):
        for j in range(CH):
            pltpu.async_copy(emb_in.at[pk_v.at[b, j, 0]],
                             rows_v.at[b, pl.ds(j * 128, 128)], sem_g[b])

    def drain_gathers(b):
        pltpu.make_async_copy(emb_in.at[pl.ds(0, C)], rows_v.at[b],
                              sem_g[b]).wait()

    def scale_rows(b):
        @plsc.parallel_loop(0, C // L, unroll=2)
        def _scale(g):
            j = g // (128 // L)
            t = g % (128 // L)
            vv = plsc.bitcast(pk_v[b, j, 2, pl.ds(t * L, L)], jnp.float32)
            base = g * L
            for i in range(L):
                rows_v[b, base + i] = rows_v[b, base + i] * vv[i]

    def fire_scatters(b):
        for j in range(CH):
            pltpu.async_copy(rows_v.at[b, pl.ds(j * 128, 128)],
                             acc.at[pk_v.at[b, j, 1]], sem_s[b], add=True)

    def drain_scatters(b):
        pltpu.make_async_copy(rows_v.at[b], acc.at[pl.ds(0, C)],
                              sem_s[b]).wait()

    load_edges(0, 0)
    drain_edges(0)
    adjust_src(0)
    fire_gathers(0)
    load_edges(1, 1)
    drain_edges(1)
    adjust_src(1)
    fire_gathers(1)

    def outer_body(k3, carry):
        for b in (0, 1, 2):
            k = k3 * 3 + b
            b1 = (b + 1) % 3
            b2 = (b + 2) % 3

            @pl.when(k >= 1)
            def _():
                drain_scatters(b2)

            @pl.when(k < NCHUNK - 2)
            def _():
                load_edges(k + 2, b2)

            drain_gathers(b)
            scale_rows(b)
            fire_scatters(b)

            @pl.when(k < NCHUNK - 2)
            def _():
                drain_edges(b2)
                adjust_src(b2)
                fire_gathers(b2)
        return carry

    lax.fori_loop(0, NCHUNK // 3, outer_body, 0)
    drain_scatters((NCHUNK - 1) % 3)
    plsc.subcore_barrier()

    pltpu.sync_copy(acc.at[pl.ds(s * ROWS_T, ROWS_T)],
                    emb_out.at[pl.ds(tab_base + s * ROWS_T, ROWS_T)])


_UB = B // NS


@functools.partial(
    pl.kernel,
    out_type=(jax.ShapeDtypeStruct((B, HDIM), jnp.float32),
              jax.ShapeDtypeStruct((B, HDIM), jnp.float32)),
    mesh=_mesh,
    scratch_types=[
        pltpu.VMEM((1, _UB), jnp.int32),
        pltpu.VMEM((_UB, HDIM), jnp.float32),
        pltpu.VMEM((_UB, HDIM), jnp.float32),
        pltpu.SemaphoreType.DMA,
    ],
    compiler_params=pltpu.CompilerParams(use_tc_tiling_on_sc=False, needs_layout_passes=False),
)
def _sc_users(e0, e1, e2, e3, inst2, out_lo, out_hi,
              idx_v, rows_v, uacc_v, sem):
    c = lax.axis_index("c")
    s = lax.axis_index("s")
    pltpu.sync_copy(inst2.at[s], idx_v)

    for t in range(_UB // L):
        sl = pl.ds(t * L, L)
        idx_v[0, sl] = idx_v[0, sl] + c * N_PAD

    pltpu.async_copy(e0.at[idx_v.at[0]], uacc_v, sem).wait()
    for e in (e1, e2, e3):
        pltpu.async_copy(e.at[idx_v.at[0]], rows_v, sem).wait()

        @plsc.parallel_loop(0, _UB, unroll=8)
        def _acc(r):
            uacc_v[r] = uacc_v[r] + rows_v[r]

    @plsc.parallel_loop(0, _UB, unroll=8)
    def _mean(r):
        uacc_v[r] = uacc_v[r] * 0.25

    @pl.when(c == 0)
    def _():
        pltpu.sync_copy(uacc_v, out_lo.at[pl.ds(s * _UB, _UB)])

    @pl.when(c == 1)
    def _():
        pltpu.sync_copy(uacc_v, out_hi.at[pl.ds(s * _UB, _UB)])


@functools.partial(
    pl.kernel,
    out_type=(jax.ShapeDtypeStruct((M_PAD, HDIM), jnp.float32),
              jax.ShapeDtypeStruct((M_PAD, HDIM), jnp.float32)),
    mesh=_mesh,
    scratch_types=[
        pltpu.VMEM((IT_T, HDIM), jnp.float32),
        pltpu.VMEM((IT_T, HDIM), jnp.float32),
    ],
    compiler_params=pltpu.CompilerParams(use_tc_tiling_on_sc=False, needs_layout_passes=False),
)
def _sc_items(e0, e1, e2, e3, out_lo, out_hi, iacc_v, in_v):
    c = lax.axis_index("c")
    s = lax.axis_index("s")
    base = c * N_PAD + N_USER + s * IT_T

    pltpu.sync_copy(e0.at[pl.ds(base, IT_T)], iacc_v)
    for e in (e1, e2, e3):
        pltpu.sync_copy(e.at[pl.ds(base, IT_T)], in_v)

        @plsc.parallel_loop(0, IT_T, unroll=8)
        def _acc(r):
            iacc_v[r] = iacc_v[r] + in_v[r]

    @plsc.parallel_loop(0, IT_T, unroll=8)
    def _mean(r):
        iacc_v[r] = iacc_v[r] * 0.25

    @pl.when(c == 0)
    def _():
        pltpu.sync_copy(iacc_v, out_lo.at[pl.ds(s * IT_T, IT_T)])

    @pl.when(c == 1)
    def _():
        pltpu.sync_copy(iacc_v, out_hi.at[pl.ds(s * IT_T, IT_T)])


BI = 2048


def _mm_body(il_ref, ih_ref, ul_ref, uh_ref, o_ref):
    acc = lax.dot_general(il_ref[...], ul_ref[...], (((1,), (1,)), ((), ())),
                          preferred_element_type=jnp.float32)
    acc += lax.dot_general(ih_ref[...], uh_ref[...], (((1,), (1,)), ((), ())),
                           preferred_element_type=jnp.float32)
    o_ref[...] = 1.0 / (1.0 + jnp.exp(-acc))


_ratings_call = pl.pallas_call(
    _mm_body,
    grid=((M_ITEM + BI - 1) // BI,),
    in_specs=[
        pl.BlockSpec((BI, HDIM), lambda i: (i, 0)),
        pl.BlockSpec((BI, HDIM), lambda i: (i, 0)),
        pl.BlockSpec((B, HDIM), lambda i: (0, 0)),
        pl.BlockSpec((B, HDIM), lambda i: (0, 0)),
    ],
    out_specs=pl.BlockSpec((BI, B), lambda i: (i, 0)),
    out_shape=jax.ShapeDtypeStruct((M_ITEM, B), jnp.float32),
)


def kernel(instances, edge_index, edge_vals, user_emb, item_emb):
    src = edge_index[0].astype(jnp.int32)
    dst = edge_index[1].astype(jnp.int32)
    valbits = lax.bitcast_convert_type(edge_vals.astype(jnp.float32),
                                       jnp.int32)

    pad = E_PAD - E
    zpad_e = jnp.zeros((pad,), jnp.int32)

    def groups(a):
        return jnp.concatenate([a, zpad_e]).reshape(-1, 128)

    packed = jnp.stack([groups(src), groups(dst), groups(valbits)], axis=1)

    all_emb = jnp.concatenate([user_emb, item_emb], axis=0)
    zpad = jnp.zeros((N_PAD - N, HDIM), jnp.float32)
    e0 = jnp.concatenate(
        [all_emb[:, :HDIM], zpad, all_emb[:, HDIM:], zpad], axis=0)

    zrows = jnp.zeros((ROWS_T, HDIM), jnp.float32)
    e1 = _sc_layer(e0, packed, zrows)
    e2 = _sc_layer(e1, packed, zrows)
    e3 = _sc_layer(e2, packed, zrows)

    inst2 = instances.astype(jnp.int32).reshape(NS, 1, _UB)
    u_lo, u_hi = _sc_users(e0, e1, e2, e3, inst2)
    items_lo, items_hi = _sc_items(e0, e1, e2, e3)

    return _ratings_call(items_lo, items_hi, u_lo, u_hi).T
